# baseline (speedup 1.0000x reference)
import jax
import jax.numpy as jnp
from jax.experimental import pallas as pl
from jax.experimental.pallas import tpu as pltpu

_BN = 512
_CH = 128


def _vq_argmin_kernel(x_ref, c_ref, o_ref):
    x = x_ref[...]
    k = c_ref.shape[0]
    bn = x.shape[0]
    nch = k // _CH
    xm2 = x * (-2.0)
    x2 = jnp.sum(x * x, axis=1, keepdims=True)
    c = c_ref[...]
    c2all = jnp.sum(c * c, axis=1)[None, :]

    def tile(j):
        cj = c_ref[pl.ds(j * _CH, _CH), :]
        ct = jax.lax.dot_general(
            xm2, cj, (((1,), (1,)), ((), ())),
            preferred_element_type=jnp.float32,
        )
        c2 = c2all[:, j * _CH:(j + 1) * _CH]
        return (x2 + c2) + ct

    run_m = tile(0)
    for j in range(1, nch):
        run_m = jnp.minimum(run_m, tile(j))
    m = jnp.min(run_m, axis=1, keepdims=True)
    v0 = jnp.maximum(m, 0.0)
    s_min = jnp.sqrt(v0)
    bits = jax.lax.bitcast_convert_type(v0, jnp.int32)
    probe = jax.lax.broadcasted_iota(jnp.int32, (bn, 8), 1)
    cand = jax.lax.bitcast_convert_type(bits + probe, jnp.float32)
    eq = jnp.sqrt(cand) == s_min
    hi = jnp.max(jnp.where(eq, cand, jnp.float32(-jnp.inf)), axis=1,
                 keepdims=True)
    run_c = None
    for j in range(nch):
        cj_id = jnp.where(tile(j) <= hi, jnp.float32(j), jnp.float32(nch))
        run_c = cj_id if run_c is None else jnp.minimum(run_c, cj_id)
    lane_f = jax.lax.broadcasted_iota(
        jnp.int32, (bn, _CH), 1).astype(jnp.float32)
    kf = run_c * jnp.float32(_CH) + lane_f
    o_ref[...] = jnp.min(kf, axis=1).astype(jnp.int32)


def kernel(x, centroids):
    n, d = x.shape
    k, _ = centroids.shape
    grid = (n // _BN,)
    return pl.pallas_call(
        _vq_argmin_kernel,
        grid=grid,
        in_specs=[
            pl.BlockSpec((_BN, d), lambda i: (i, 0)),
            pl.BlockSpec((k, d), lambda i: (0, 0)),
        ],
        out_specs=pl.BlockSpec((_BN,), lambda i: (i,)),
        out_shape=jax.ShapeDtypeStruct((n,), jnp.int32),
        compiler_params=pltpu.CompilerParams(
            dimension_semantics=("parallel",),
        ),
    )(x, centroids)


# speedup vs baseline: 1.0144x; 1.0144x over previous
import jax
import jax.numpy as jnp
from jax.experimental import pallas as pl
from jax.experimental.pallas import tpu as pltpu

_BN = 512
_CH = 128


def _vq_argmin_kernel(x_ref, c_ref, o_ref):
    x = x_ref[...]
    k = c_ref.shape[0]
    bn = x.shape[0]
    nch = k // _CH
    xm2 = x * (-2.0)
    x2 = jnp.sum(x * x, axis=1, keepdims=True)
    c = c_ref[...]
    c2all = jnp.sum(c * c, axis=1)[None, :]

    def tile(j):
        cj = c_ref[pl.ds(j * _CH, _CH), :]
        ct = jax.lax.dot_general(
            xm2, cj, (((1,), (1,)), ((), ())),
            preferred_element_type=jnp.float32,
        )
        c2 = c2all[:, j * _CH:(j + 1) * _CH]
        return (x2 + c2) + ct

    run_m = tile(0)
    for j in range(1, nch):
        run_m = jnp.minimum(run_m, tile(j))
    m = jnp.min(run_m, axis=1, keepdims=True)
    v0 = jnp.maximum(m, 0.0)
    s_min = jnp.sqrt(v0)
    bits = jax.lax.bitcast_convert_type(v0, jnp.int32)
    probe = jax.lax.broadcasted_iota(jnp.int32, (bn, 8), 1)
    cand = jax.lax.bitcast_convert_type(bits + probe, jnp.float32)
    eq = jnp.sqrt(cand) == s_min
    hi = jnp.max(jnp.where(eq, cand, jnp.float32(-jnp.inf)), axis=1,
                 keepdims=True)
    lane_f = jax.lax.broadcasted_iota(
        jnp.int32, (bn, _CH), 1).astype(jnp.float32)
    run_i = None
    for j in range(nch):
        idx = jnp.where(tile(j) <= hi, lane_f + jnp.float32(j * _CH),
                        jnp.float32(k))
        run_i = idx if run_i is None else jnp.minimum(run_i, idx)
    o_ref[...] = jnp.min(run_i, axis=1).astype(jnp.int32)


def kernel(x, centroids):
    n, d = x.shape
    k, _ = centroids.shape
    grid = (n // _BN,)
    return pl.pallas_call(
        _vq_argmin_kernel,
        grid=grid,
        in_specs=[
            pl.BlockSpec((_BN, d), lambda i: (i, 0)),
            pl.BlockSpec((k, d), lambda i: (0, 0)),
        ],
        out_specs=pl.BlockSpec((_BN,), lambda i: (i,)),
        out_shape=jax.ShapeDtypeStruct((n,), jnp.int32),
        compiler_params=pltpu.CompilerParams(
            dimension_semantics=("arbitrary",),
        ),
    )(x, centroids)
